# Initial kernel scaffold; baseline (speedup 1.0000x reference)
import jax
import jax.numpy as jnp
from jax.experimental import pallas as pl


def kernel(imgs, params, feats, labels):
    raise NotImplementedError("write your pallas kernel here")


# speedup vs baseline: 5.0925x; 5.0925x over previous
import jax
import jax.numpy as jnp
from jax import lax
from jax.experimental import pallas as pl
from jax.experimental.pallas import tpu as pltpu

_CFG = [64, 'M', 128, 'M', 256, 256, 'M', 512, 512, 'M', 512, 512, 'M']
_N = 20000
_K = 29
_NCLS = 10
_NPAD = 20480
_BLK = 2048
_BIG = 1e30


def _features(x, params):
    i = 0
    for c in _CFG:
        if c == 'M':
            x = lax.reduce_window(x, -jnp.inf, lax.max, (1, 1, 2, 2), (1, 1, 2, 2), 'VALID')
        else:
            w = params['w%d' % i]
            b = params['b%d' % i]
            x = lax.conv_general_dilated(x, w, (1, 1), 'SAME',
                                         dimension_numbers=('NCHW', 'OIHW', 'NCHW'))
            x = x + b[None, :, None, None]
            x = params['g%d' % i][None, :, None, None] * (x / jnp.sqrt(jnp.float32(1.0 + 1e-5))) \
                + params['be%d' % i][None, :, None, None]
            x = jax.nn.relu(x)
            i += 1
    return x


def _retrieval_body(fT_ref, feats_ref, labels_ref, out_ref, D_ref):
    i = pl.program_id(0)
    nblk = pl.num_programs(0)

    blk = feats_ref[...]
    blkT = jnp.swapaxes(blk, 0, 1)
    col = lax.broadcasted_iota(jnp.int32, (1, _BLK), 1) + i * _BLK
    valid = col < _N

    rows = []
    for q in range(8):
        fq = fT_ref[:, q:q + 1]
        d = jnp.sum(jnp.abs(blkT - fq), axis=0, keepdims=True)
        rows.append(jnp.where(valid, d, _BIG))
    D_ref[:, pl.ds(i * _BLK, _BLK)] = jnp.concatenate(rows, axis=0)

    @pl.when(i == nblk - 1)
    def _():
        iota = lax.broadcasted_iota(jnp.int32, (8, _NPAD), 1)
        labs = jnp.broadcast_to(labels_ref[...], (8, _NPAD))
        kio = lax.broadcasted_iota(jnp.int32, (8, 32), 1)

        def step(k, carry):
            dvals, dlabs = carry
            D = D_ref[...]
            m = jnp.min(D, axis=1, keepdims=True)
            idx = jnp.min(jnp.where(D == m, iota, _NPAD), axis=1, keepdims=True)
            hit = iota == idx
            lab = jnp.max(jnp.where(hit, labs, -1), axis=1, keepdims=True)
            dvals = jnp.where(kio == k, m, dvals)
            dlabs = jnp.where(kio == k, lab, dlabs)
            D_ref[...] = jnp.where(hit, _BIG, D)
            return dvals, dlabs

        dvals0 = jnp.full((8, 32), _BIG, jnp.float32)
        dlabs0 = jnp.full((8, 32), -1, jnp.int32)
        dvals, dlabs = lax.fori_loop(0, _K, step, (dvals0, dlabs0))

        s = dvals[:, 14:15]
        kvalid = kio < _K
        e = jnp.where(kvalid, jnp.exp(-dvals / s), 0.0)
        counts = jnp.concatenate(
            [jnp.sum(jnp.where(kvalid & (dlabs == c), 1, 0), axis=1, keepdims=True)
             for c in range(_NCLS)], axis=1)
        maxc = jnp.max(counts, axis=1, keepdims=True)
        cio = lax.broadcasted_iota(jnp.int32, (8, _NCLS), 1)
        pred = jnp.min(jnp.where(counts == maxc, cio, _NCLS), axis=1, keepdims=True)
        nr = jnp.sum(jnp.where(dlabs == pred, e, 0.0), axis=1, keepdims=True)
        dr = jnp.sum(e, axis=1, keepdims=True)
        p = nr / dr
        out_ref[...] = jnp.concatenate([p, 1.0 - p], axis=1)


def _retrieval(fT, feats, labels2d, interpret=False):
    return pl.pallas_call(
        _retrieval_body,
        grid=(_NPAD // _BLK,),
        in_specs=[
            pl.BlockSpec((512, 8), lambda i: (0, 0)),
            pl.BlockSpec((_BLK, 512), lambda i: (i, 0)),
            pl.BlockSpec((1, _NPAD), lambda i: (0, 0)),
        ],
        out_specs=pl.BlockSpec((8, 2), lambda i: (0, 0)),
        out_shape=jax.ShapeDtypeStruct((8, 2), jnp.float32),
        scratch_shapes=[pltpu.VMEM((8, _NPAD), jnp.float32)],
        interpret=interpret,
    )(fT, feats, labels2d)


def kernel(imgs, params, feats, labels):
    f = _features(imgs, params).reshape(imgs.shape[0], -1)
    fT = f.T
    labels2d = jnp.pad(labels, (0, _NPAD - _N)).reshape(1, _NPAD)
    return _retrieval(fT, feats, labels2d)
